# Initial kernel scaffold; baseline (speedup 1.0000x reference)
import jax
import jax.numpy as jnp
from jax.experimental import pallas as pl


def kernel(x, edge_index, batch, params):
    raise NotImplementedError("write your pallas kernel here")


# speedup vs baseline: 4.3532x; 4.3532x over previous
import functools

import jax
import jax.numpy as jnp
from jax import lax
from jax.experimental import pallas as pl
from jax.experimental.pallas import tpu as pltpu
from jax.experimental.pallas import tpu_sc as plsc

_N = 10000
_E = 320000
_D = 128
_G = 64
_NC = 2
_NS = 16
_NW = _NC * _NS
_EPW = _E // _NW
_CHUNK = 80
_NCH = _EPW // _CHUNK
_RCH = 80
_NRC = _N // _RCH
_BN = 1000


def _segsum_sc(h, src, dst):
    mesh = plsc.VectorSubcoreMesh(core_axis_name="c", subcore_axis_name="s")

    @functools.partial(
        pl.kernel,
        mesh=mesh,
        out_type=jax.ShapeDtypeStruct((_NC, _N, _D), jnp.float32),
        scratch_types=[
            pltpu.VMEM((_CHUNK,), jnp.int32),
            pltpu.VMEM((_CHUNK,), jnp.int32),
            pltpu.VMEM((_CHUNK, _D), jnp.float32),
            pltpu.VMEM_SHARED((_N, _D), jnp.float32),
            pltpu.SemaphoreType.DMA,
        ],
    )
    def k(h_hbm, src_hbm, dst_hbm, out_hbm, src_v, dst_v, rows_v,
          acc_sh, sem):
        cid = lax.axis_index("c")
        sid = lax.axis_index("s")
        wid = cid * _NS + sid

        def zb(i, _):
            rows_v[i // 8, pl.ds((i % 8) * 16, 16)] = jnp.zeros(
                (16,), jnp.float32)
            return 0
        lax.fori_loop(0, _RCH * 8, zb, 0)

        def zc(j, _):
            c = sid + j * _NS

            @pl.when(c < _NRC)
            def _():
                pltpu.sync_copy(rows_v, acc_sh.at[pl.ds(c * _RCH, _RCH)])
            return 0
        lax.fori_loop(0, (_NRC + _NS - 1) // _NS, zc, 0)
        plsc.subcore_barrier()

        def body(ch, _):
            base = wid * _EPW + ch * _CHUNK
            pltpu.sync_copy(src_hbm.at[pl.ds(base, _CHUNK)], src_v)
            pltpu.sync_copy(dst_hbm.at[pl.ds(base, _CHUNK)], dst_v)
            pltpu.async_copy(h_hbm.at[src_v], rows_v, sem).wait()
            pltpu.sync_copy(rows_v, acc_sh.at[dst_v], add=True)
            return 0
        lax.fori_loop(0, _NCH, body, 0)
        plsc.subcore_barrier()

        def dr(j, _):
            c = sid + j * _NS

            @pl.when(c < _NRC)
            def _():
                off = c * _RCH
                pltpu.sync_copy(acc_sh.at[pl.ds(off, _RCH)], rows_v)
                pltpu.sync_copy(rows_v, out_hbm.at[cid, pl.ds(off, _RCH)])
            return 0
        lax.fori_loop(0, (_NRC + _NS - 1) // _NS, dr, 0)

    return k(h, src, dst)


def _row_spec(shape=(_BN, _D)):
    return pl.BlockSpec(shape, lambda i: (i, 0))


def _full_spec(shape):
    return pl.BlockSpec(shape, lambda i: tuple(0 for _ in shape))


def _mlp_stage1(h, p0, p1, w1, b1):
    def kern(h_ref, p0_ref, p1_ref, w1_ref, b1_ref, t_ref, s_ref):
        z = h_ref[...] + p0_ref[...] + p1_ref[...]
        t = jnp.dot(z, w1_ref[...], preferred_element_type=jnp.float32)
        t = t + b1_ref[...]
        t_ref[...] = t

        @pl.when(pl.program_id(0) == 0)
        def _():
            s_ref[...] = jnp.zeros_like(s_ref)

        s_ref[...] += jnp.sum(t, axis=0, keepdims=True)

    return pl.pallas_call(
        kern,
        grid=(_N // _BN,),
        in_specs=[
            _row_spec(), _row_spec(), _row_spec(),
            _full_spec((_D, _D)), _full_spec((1, _D)),
        ],
        out_specs=[
            _row_spec(),
            _full_spec((1, _D)),
        ],
        out_shape=[
            jax.ShapeDtypeStruct((_N, _D), jnp.float32),
            jax.ShapeDtypeStruct((1, _D), jnp.float32),
        ],
    )(h, p0, p1, w1, b1.reshape(1, _D))


def _colvar(t, s):
    def kern(t_ref, s_ref, q_ref):
        d = t_ref[...] - s_ref[...] * (1.0 / _N)

        @pl.when(pl.program_id(0) == 0)
        def _():
            q_ref[...] = jnp.zeros_like(q_ref)

        q_ref[...] += jnp.sum(d * d, axis=0, keepdims=True)

    return pl.pallas_call(
        kern,
        grid=(_N // _BN,),
        in_specs=[_row_spec(), _full_spec((1, _D))],
        out_specs=_full_spec((1, _D)),
        out_shape=jax.ShapeDtypeStruct((1, _D), jnp.float32),
    )(t, s)


def _bn_relu(t, s_ref, q_ref, g_ref, be_ref):
    mean = s_ref[...] * (1.0 / _N)
    var = q_ref[...] * (1.0 / _N)
    inv = lax.rsqrt(var + 1e-5) * g_ref[...]
    return jnp.maximum((t - mean) * inv + be_ref[...], 0.0)


def _mlp_stage2(t, s, q, gamma, beta, w2, b2):
    def kern(t_ref, s_ref, q_ref, g_ref, be_ref, w2_ref, b2_ref, o_ref):
        nrm = _bn_relu(t_ref[...], s_ref, q_ref, g_ref, be_ref)
        y = jnp.dot(nrm, w2_ref[...], preferred_element_type=jnp.float32)
        o_ref[...] = jnp.maximum(y + b2_ref[...], 0.0)

    return pl.pallas_call(
        kern,
        grid=(_N // _BN,),
        in_specs=[
            _row_spec(),
            _full_spec((1, _D)), _full_spec((1, _D)),
            _full_spec((1, _D)), _full_spec((1, _D)),
            _full_spec((_D, _D)), _full_spec((1, _D)),
        ],
        out_specs=_row_spec(),
        out_shape=jax.ShapeDtypeStruct((_N, _D), jnp.float32),
    )(t, s, q, gamma.reshape(1, _D), beta.reshape(1, _D), w2,
      b2.reshape(1, _D))


def _mlp_stage2_pool(t, s, q, gamma, beta, w2, b2, batch3d):
    def kern(t_ref, s_ref, q_ref, g_ref, be_ref, w2_ref, b2_ref, b_ref,
             ps_ref, ct_ref):
        nrm = _bn_relu(t_ref[...], s_ref, q_ref, g_ref, be_ref)
        y = jnp.dot(nrm, w2_ref[...], preferred_element_type=jnp.float32)
        h3 = jnp.maximum(y + b2_ref[...], 0.0)
        b = b_ref[...][0, 0]
        gid = lax.broadcasted_iota(jnp.int32, (_G, _BN), 0)
        oh = (b[None, :] == gid).astype(jnp.float32)

        @pl.when(pl.program_id(0) == 0)
        def _():
            ps_ref[...] = jnp.zeros_like(ps_ref)
            ct_ref[...] = jnp.zeros_like(ct_ref)

        ps_ref[...] += jnp.dot(oh, h3, preferred_element_type=jnp.float32)
        ct_ref[...] += jnp.sum(oh, axis=1, keepdims=True)

    return pl.pallas_call(
        kern,
        grid=(_N // _BN,),
        in_specs=[
            _row_spec(),
            _full_spec((1, _D)), _full_spec((1, _D)),
            _full_spec((1, _D)), _full_spec((1, _D)),
            _full_spec((_D, _D)), _full_spec((1, _D)),
            pl.BlockSpec((1, 1, _BN), lambda i: (i, 0, 0)),
        ],
        out_specs=[
            _full_spec((_G, _D)),
            _full_spec((_G, 1)),
        ],
        out_shape=[
            jax.ShapeDtypeStruct((_G, _D), jnp.float32),
            jax.ShapeDtypeStruct((_G, 1), jnp.float32),
        ],
    )(t, s, q, gamma.reshape(1, _D), beta.reshape(1, _D), w2,
      b2.reshape(1, _D), batch3d)


def _final_mlp(ps, ct, p):
    def kern(ps_ref, ct_ref, w1_ref, b1_ref, g_ref, be_ref, w2_ref, b2_ref,
             o_ref):
        pooled = ps_ref[...] / jnp.maximum(ct_ref[...], 1.0)
        t = jnp.dot(pooled, w1_ref[...], preferred_element_type=jnp.float32)
        t = t + b1_ref[...]
        mean = jnp.mean(t, axis=0, keepdims=True)
        var = jnp.mean((t - mean) * (t - mean), axis=0, keepdims=True)
        nrm = (t - mean) * lax.rsqrt(var + 1e-5) * g_ref[...] + be_ref[...]
        nrm = jnp.maximum(nrm, 0.0)
        y = jnp.dot(nrm, w2_ref[...], preferred_element_type=jnp.float32)
        o_ref[...] = y + b2_ref[...]

    return pl.pallas_call(
        kern,
        grid=(1,),
        in_specs=[
            _full_spec((_G, _D)), _full_spec((_G, 1)),
            _full_spec((_D, _D)), _full_spec((1, _D)),
            _full_spec((1, _D)), _full_spec((1, _D)),
            _full_spec((_D, _D)), _full_spec((1, _D)),
        ],
        out_specs=_full_spec((_G, _D)),
        out_shape=jax.ShapeDtypeStruct((_G, _D), jnp.float32),
    )(ps, ct, p['W1'], p['b1'].reshape(1, _D), p['gamma'].reshape(1, _D),
      p['beta'].reshape(1, _D), p['W2'], p['b2'].reshape(1, _D))


def kernel(x, edge_index, batch, params):
    src = edge_index[0]
    dst = edge_index[1]
    batch3d = batch.reshape(_N // _BN, 1, _BN)
    h = x
    for i in range(3):
        p = _segsum_sc(h, src, dst)
        t, s = _mlp_stage1(h, p[0], p[1], params[i]['W1'], params[i]['b1'])
        q = _colvar(t, s)
        if i < 2:
            h = _mlp_stage2(t, s, q, params[i]['gamma'], params[i]['beta'],
                            params[i]['W2'], params[i]['b2'])
        else:
            ps, ct = _mlp_stage2_pool(t, s, q, params[i]['gamma'],
                                      params[i]['beta'], params[i]['W2'],
                                      params[i]['b2'], batch3d)
    return _final_mlp(ps, ct, params[3])


# speedup vs baseline: 7.3726x; 1.6936x over previous
import functools

import jax
import jax.numpy as jnp
from jax import lax
from jax.experimental import pallas as pl
from jax.experimental.pallas import tpu as pltpu
from jax.experimental.pallas import tpu_sc as plsc

_N = 10000
_E = 320000
_D = 128
_G = 64
_NC = 2
_NS = 16
_NW = _NC * _NS
_EPW = _E // _NW
_CHUNK = 80
_NCH = _EPW // _CHUNK
_RCH = 80
_IBK = 25
_NBK = _NCH // _IBK
_NRC = _N // _RCH
_BN = 1000


def _segsum_sc(h, src3d, dst3d):
    mesh = plsc.VectorSubcoreMesh(core_axis_name="c", subcore_axis_name="s")

    @functools.partial(
        pl.kernel,
        mesh=mesh,
        out_type=jax.ShapeDtypeStruct((_NC, _N, _D), jnp.float32),
        scratch_types=[
            pltpu.VMEM((_IBK, _CHUNK), jnp.int32),
            pltpu.VMEM((_IBK, _CHUNK), jnp.int32),
            pltpu.VMEM((_CHUNK, _D), jnp.float32),
            pltpu.VMEM((_CHUNK, _D), jnp.float32),
            pltpu.VMEM_SHARED((_N, _D), jnp.float32),
            pltpu.SemaphoreType.DMA,
            pltpu.SemaphoreType.DMA,
        ],
    )
    def k(h_hbm, src_hbm, dst_hbm, out_hbm, src_v, dst_v, rows_a, rows_b,
          acc_sh, sem_a, sem_b):
        cid = lax.axis_index("c")
        sid = lax.axis_index("s")
        wid = cid * _NS + sid

        def zb(i, _):
            rows_a[i // 8, pl.ds((i % 8) * 16, 16)] = jnp.zeros(
                (16,), jnp.float32)
            return 0
        lax.fori_loop(0, _RCH * 8, zb, 0)

        def zc(j, _):
            c = sid + j * _NS

            @pl.when(c < _NRC)
            def _():
                pltpu.sync_copy(rows_a, acc_sh.at[pl.ds(c * _RCH, _RCH)])
            return 0
        lax.fori_loop(0, (_NRC + _NS - 1) // _NS, zc, 0)
        plsc.subcore_barrier()

        def blk(b, _):
            pltpu.sync_copy(src_hbm.at[wid, b], src_v)
            pltpu.sync_copy(dst_hbm.at[wid, b], dst_v)
            pltpu.async_copy(h_hbm.at[src_v.at[0]], rows_a, sem_a)

            def body(c, _):
                def step(rows_cur, sem_cur, rows_nxt, sem_nxt):
                    pltpu.make_async_copy(h_hbm.at[src_v.at[c]], rows_cur,
                                          sem_cur).wait()

                    @pl.when(c + 1 < _IBK)
                    def _():
                        pltpu.async_copy(h_hbm.at[src_v.at[c + 1]], rows_nxt,
                                         sem_nxt)
                    pltpu.sync_copy(rows_cur, acc_sh.at[dst_v.at[c]],
                                    add=True)

                @pl.when(c % 2 == 0)
                def _():
                    step(rows_a, sem_a, rows_b, sem_b)

                @pl.when(c % 2 == 1)
                def _():
                    step(rows_b, sem_b, rows_a, sem_a)
                return 0
            lax.fori_loop(0, _IBK, body, 0)
            return 0
        lax.fori_loop(0, _NBK, blk, 0)
        plsc.subcore_barrier()

        def dr(j, _):
            c = sid + j * _NS

            @pl.when(c < _NRC)
            def _():
                off = c * _RCH
                pltpu.sync_copy(acc_sh.at[pl.ds(off, _RCH)], rows_a)
                pltpu.sync_copy(rows_a, out_hbm.at[cid, pl.ds(off, _RCH)])
            return 0
        lax.fori_loop(0, (_NRC + _NS - 1) // _NS, dr, 0)

    return k(h, src3d, dst3d)


def _row_spec(shape=(_BN, _D)):
    return pl.BlockSpec(shape, lambda i: (i, 0))


def _full_spec(shape):
    return pl.BlockSpec(shape, lambda i: tuple(0 for _ in shape))


def _mlp_stage1(h, p0, p1, w1, b1):
    def kern(h_ref, p0_ref, p1_ref, w1_ref, b1_ref, t_ref, s_ref):
        z = h_ref[...] + p0_ref[...] + p1_ref[...]
        t = jnp.dot(z, w1_ref[...], preferred_element_type=jnp.float32)
        t = t + b1_ref[...]
        t_ref[...] = t

        @pl.when(pl.program_id(0) == 0)
        def _():
            s_ref[...] = jnp.zeros_like(s_ref)

        s_ref[...] += jnp.sum(t, axis=0, keepdims=True)

    return pl.pallas_call(
        kern,
        grid=(_N // _BN,),
        in_specs=[
            _row_spec(), _row_spec(), _row_spec(),
            _full_spec((_D, _D)), _full_spec((1, _D)),
        ],
        out_specs=[
            _row_spec(),
            _full_spec((1, _D)),
        ],
        out_shape=[
            jax.ShapeDtypeStruct((_N, _D), jnp.float32),
            jax.ShapeDtypeStruct((1, _D), jnp.float32),
        ],
    )(h, p0, p1, w1, b1.reshape(1, _D))


def _colvar(t, s):
    def kern(t_ref, s_ref, q_ref):
        d = t_ref[...] - s_ref[...] * (1.0 / _N)

        @pl.when(pl.program_id(0) == 0)
        def _():
            q_ref[...] = jnp.zeros_like(q_ref)

        q_ref[...] += jnp.sum(d * d, axis=0, keepdims=True)

    return pl.pallas_call(
        kern,
        grid=(_N // _BN,),
        in_specs=[_row_spec(), _full_spec((1, _D))],
        out_specs=_full_spec((1, _D)),
        out_shape=jax.ShapeDtypeStruct((1, _D), jnp.float32),
    )(t, s)


def _bn_relu(t, s_ref, q_ref, g_ref, be_ref):
    mean = s_ref[...] * (1.0 / _N)
    var = q_ref[...] * (1.0 / _N)
    inv = lax.rsqrt(var + 1e-5) * g_ref[...]
    return jnp.maximum((t - mean) * inv + be_ref[...], 0.0)


def _mlp_stage2(t, s, q, gamma, beta, w2, b2):
    def kern(t_ref, s_ref, q_ref, g_ref, be_ref, w2_ref, b2_ref, o_ref):
        nrm = _bn_relu(t_ref[...], s_ref, q_ref, g_ref, be_ref)
        y = jnp.dot(nrm, w2_ref[...], preferred_element_type=jnp.float32)
        o_ref[...] = jnp.maximum(y + b2_ref[...], 0.0)

    return pl.pallas_call(
        kern,
        grid=(_N // _BN,),
        in_specs=[
            _row_spec(),
            _full_spec((1, _D)), _full_spec((1, _D)),
            _full_spec((1, _D)), _full_spec((1, _D)),
            _full_spec((_D, _D)), _full_spec((1, _D)),
        ],
        out_specs=_row_spec(),
        out_shape=jax.ShapeDtypeStruct((_N, _D), jnp.float32),
    )(t, s, q, gamma.reshape(1, _D), beta.reshape(1, _D), w2,
      b2.reshape(1, _D))


def _mlp_stage2_pool(t, s, q, gamma, beta, w2, b2, batch3d):
    def kern(t_ref, s_ref, q_ref, g_ref, be_ref, w2_ref, b2_ref, b_ref,
             ps_ref, ct_ref):
        nrm = _bn_relu(t_ref[...], s_ref, q_ref, g_ref, be_ref)
        y = jnp.dot(nrm, w2_ref[...], preferred_element_type=jnp.float32)
        h3 = jnp.maximum(y + b2_ref[...], 0.0)
        b = b_ref[...][0, 0]
        gid = lax.broadcasted_iota(jnp.int32, (_G, _BN), 0)
        oh = (b[None, :] == gid).astype(jnp.float32)

        @pl.when(pl.program_id(0) == 0)
        def _():
            ps_ref[...] = jnp.zeros_like(ps_ref)
            ct_ref[...] = jnp.zeros_like(ct_ref)

        ps_ref[...] += jnp.dot(oh, h3, preferred_element_type=jnp.float32)
        ct_ref[...] += jnp.sum(oh, axis=1, keepdims=True)

    return pl.pallas_call(
        kern,
        grid=(_N // _BN,),
        in_specs=[
            _row_spec(),
            _full_spec((1, _D)), _full_spec((1, _D)),
            _full_spec((1, _D)), _full_spec((1, _D)),
            _full_spec((_D, _D)), _full_spec((1, _D)),
            pl.BlockSpec((1, 1, _BN), lambda i: (i, 0, 0)),
        ],
        out_specs=[
            _full_spec((_G, _D)),
            _full_spec((_G, 1)),
        ],
        out_shape=[
            jax.ShapeDtypeStruct((_G, _D), jnp.float32),
            jax.ShapeDtypeStruct((_G, 1), jnp.float32),
        ],
    )(t, s, q, gamma.reshape(1, _D), beta.reshape(1, _D), w2,
      b2.reshape(1, _D), batch3d)


def _final_mlp(ps, ct, p):
    def kern(ps_ref, ct_ref, w1_ref, b1_ref, g_ref, be_ref, w2_ref, b2_ref,
             o_ref):
        pooled = ps_ref[...] / jnp.maximum(ct_ref[...], 1.0)
        t = jnp.dot(pooled, w1_ref[...], preferred_element_type=jnp.float32)
        t = t + b1_ref[...]
        mean = jnp.mean(t, axis=0, keepdims=True)
        var = jnp.mean((t - mean) * (t - mean), axis=0, keepdims=True)
        nrm = (t - mean) * lax.rsqrt(var + 1e-5) * g_ref[...] + be_ref[...]
        nrm = jnp.maximum(nrm, 0.0)
        y = jnp.dot(nrm, w2_ref[...], preferred_element_type=jnp.float32)
        o_ref[...] = y + b2_ref[...]

    return pl.pallas_call(
        kern,
        grid=(1,),
        in_specs=[
            _full_spec((_G, _D)), _full_spec((_G, 1)),
            _full_spec((_D, _D)), _full_spec((1, _D)),
            _full_spec((1, _D)), _full_spec((1, _D)),
            _full_spec((_D, _D)), _full_spec((1, _D)),
        ],
        out_specs=_full_spec((_G, _D)),
        out_shape=jax.ShapeDtypeStruct((_G, _D), jnp.float32),
    )(ps, ct, p['W1'], p['b1'].reshape(1, _D), p['gamma'].reshape(1, _D),
      p['beta'].reshape(1, _D), p['W2'], p['b2'].reshape(1, _D))


def kernel(x, edge_index, batch, params):
    src3d = edge_index[0].reshape(_NW, _NBK, _IBK, _CHUNK)
    dst3d = edge_index[1].reshape(_NW, _NBK, _IBK, _CHUNK)
    batch3d = batch.reshape(_N // _BN, 1, _BN)
    h = x
    for i in range(3):
        p = _segsum_sc(h, src3d, dst3d)
        t, s = _mlp_stage1(h, p[0], p[1], params[i]['W1'], params[i]['b1'])
        q = _colvar(t, s)
        if i < 2:
            h = _mlp_stage2(t, s, q, params[i]['gamma'], params[i]['beta'],
                            params[i]['W2'], params[i]['b2'])
        else:
            ps, ct = _mlp_stage2_pool(t, s, q, params[i]['gamma'],
                                      params[i]['beta'], params[i]['W2'],
                                      params[i]['b2'], batch3d)
    return _final_mlp(ps, ct, params[3])
